# baseline (speedup 1.0000x reference)
import functools

import jax
import jax.numpy as jnp
import numpy as np
from jax import lax
from jax.experimental import pallas as pl
from jax.experimental.pallas import tpu as pltpu
from jax.experimental.pallas import tpu_sc as plsc

B = 4096
L = 50
EMB = 64
NC = 2
NS = 16
NW = NC * NS
BPW = B // NW
LANES = 16
NBUF = 2

_PERM = np.concatenate([
    np.arange(0, 32, 2), np.arange(1, 32, 2),
    np.arange(32, 64, 2), np.arange(33, 64, 2)])


def _pool_kernel(x_hbm, table_hbm, out_hbm, idx_v, bufs, out_v, sems):
    wid = lax.axis_index("s") * NC + lax.axis_index("c")
    base = wid * BPW
    pltpu.sync_copy(x_hbm.at[pl.ds(base, BPW)], idx_v)

    himask = jnp.full((LANES,), 0xFFFF0000, dtype=jnp.uint32)
    sixteen = jnp.full((LANES,), 16, dtype=jnp.uint32)

    for k in range(NBUF):
        pltpu.async_copy(table_hbm.at[idx_v.at[k]], bufs[k], sems[k])

    def body(g, carry):
        for k in range(NBUF):
            b = g * NBUF + k
            buf = bufs[k]
            pltpu.make_async_copy(table_hbm.at[idx_v.at[b]], buf, sems[k]
                                  ).wait()
            accs = [jnp.zeros((LANES,), jnp.float32) for _ in range(4)]
            for l in range(L):
                for h in range(2):
                    v = plsc.bitcast(buf[l, pl.ds(h * 32, 32)], jnp.uint32)
                    lo = plsc.bitcast(v << sixteen, jnp.float32)
                    hi = plsc.bitcast(v & himask, jnp.float32)
                    accs[2 * h] = accs[2 * h] + lo
                    accs[2 * h + 1] = accs[2 * h + 1] + hi

            @pl.when(b + NBUF < BPW)
            def _():
                pltpu.async_copy(table_hbm.at[idx_v.at[b + NBUF]], buf,
                                 sems[k])

            for c in range(4):
                out_v[b, pl.ds(c * LANES, LANES)] = accs[c]
        return carry

    lax.fori_loop(0, BPW // NBUF, body, 0)
    pltpu.sync_copy(out_v, out_hbm.at[pl.ds(base, BPW)])


@functools.partial(
    pl.kernel,
    mesh=plsc.VectorSubcoreMesh(core_axis_name="c", subcore_axis_name="s"),
    out_type=jax.ShapeDtypeStruct((B, EMB), jnp.float32),
    scratch_types=[
        pltpu.VMEM((BPW, L), jnp.int32),
        *[pltpu.VMEM((L, EMB), jnp.bfloat16) for _ in range(NBUF)],
        pltpu.VMEM((BPW, EMB), jnp.float32),
        *[pltpu.SemaphoreType.DMA for _ in range(NBUF)],
    ],
    compiler_params=pltpu.CompilerParams(use_tc_tiling_on_sc=False,
                                         needs_layout_passes=False),
)
def _pool(x_hbm, table_hbm, out_hbm, idx_v, *rest):
    bufs = list(rest[:NBUF])
    out_v = rest[NBUF]
    sems = list(rest[NBUF + 1:NBUF + 1 + NBUF])
    _pool_kernel(x_hbm, table_hbm, out_hbm, idx_v, bufs, out_v, sems)


def _mlp_kernel(h_ref, w1_ref, b1_ref, w2_ref, b2_ref, w3_ref, b3_ref, o_ref):
    dn = (((1,), (1,)), ((), ()))
    h = h_ref[...]
    z = lax.dot_general(h, w1_ref[...], dn, preferred_element_type=jnp.float32)
    z = jnp.maximum(z + b1_ref[...], 0.0)
    z = lax.dot_general(z, w2_ref[...], dn, preferred_element_type=jnp.float32)
    z = jnp.maximum(z + b2_ref[...], 0.0)
    z = lax.dot_general(z, w3_ref[...], dn, preferred_element_type=jnp.float32)
    o_ref[...] = z + b3_ref[...]


def kernel(x, table, W1, b1, W2, b2, W3, b3):
    h = _pool(x, table.astype(jnp.bfloat16))
    w1p = W1[:, _PERM] * jnp.float32(1.0 / L)
    return pl.pallas_call(
        _mlp_kernel,
        out_shape=jax.ShapeDtypeStruct((B, EMB), jnp.float32),
    )(h, w1p, b1.reshape(1, -1), W2, b2.reshape(1, -1), W3, b3.reshape(1, -1))


# speedup vs baseline: 1.1443x; 1.1443x over previous
import functools

import jax
import jax.numpy as jnp
import numpy as np
from jax import lax
from jax.experimental import pallas as pl
from jax.experimental.pallas import tpu as pltpu
from jax.experimental.pallas import tpu_sc as plsc

B = 4096
L = 50
EMB = 64
NC = 2
NS = 16
NW = NC * NS
BPW = B // NW
LANES = 16
NBUF = 4

_PERM = np.concatenate([
    np.arange(0, 32, 2), np.arange(1, 32, 2),
    np.arange(32, 64, 2), np.arange(33, 64, 2)])


def _pool_kernel(x_hbm, table_hbm, out_hbm, idx_v, bufs, out_v, sems):
    wid = lax.axis_index("s") * NC + lax.axis_index("c")
    base = wid * BPW
    pltpu.sync_copy(x_hbm.at[pl.ds(base, BPW)], idx_v)

    himask = jnp.full((LANES,), 0xFFFF0000, dtype=jnp.uint32)
    sixteen = jnp.full((LANES,), 16, dtype=jnp.uint32)

    for k in range(NBUF):
        pltpu.async_copy(table_hbm.at[idx_v.at[k]], bufs[k], sems[k])

    def body(g, carry):
        for k in range(NBUF):
            b = g * NBUF + k
            buf = bufs[k]
            pltpu.make_async_copy(table_hbm.at[idx_v.at[b]], buf, sems[k]
                                  ).wait()
            accs = [jnp.zeros((LANES,), jnp.float32) for _ in range(4)]
            for l in range(L):
                for h in range(2):
                    v = plsc.bitcast(buf[l, pl.ds(h * 32, 32)], jnp.uint32)
                    lo = plsc.bitcast(v << sixteen, jnp.float32)
                    hi = plsc.bitcast(v & himask, jnp.float32)
                    accs[2 * h] = accs[2 * h] + lo
                    accs[2 * h + 1] = accs[2 * h + 1] + hi

            @pl.when(b + NBUF < BPW)
            def _():
                pltpu.async_copy(table_hbm.at[idx_v.at[b + NBUF]], buf,
                                 sems[k])

            for c in range(4):
                out_v[b, pl.ds(c * LANES, LANES)] = accs[c]
        return carry

    lax.fori_loop(0, BPW // NBUF, body, 0)
    pltpu.sync_copy(out_v, out_hbm.at[pl.ds(base, BPW)])


@functools.partial(
    pl.kernel,
    mesh=plsc.VectorSubcoreMesh(core_axis_name="c", subcore_axis_name="s"),
    out_type=jax.ShapeDtypeStruct((B, EMB), jnp.float32),
    scratch_types=[
        pltpu.VMEM((BPW, L), jnp.int32),
        *[pltpu.VMEM((L, EMB), jnp.bfloat16) for _ in range(NBUF)],
        pltpu.VMEM((BPW, EMB), jnp.float32),
        *[pltpu.SemaphoreType.DMA for _ in range(NBUF)],
    ],
    compiler_params=pltpu.CompilerParams(use_tc_tiling_on_sc=False,
                                         needs_layout_passes=False),
)
def _pool(x_hbm, table_hbm, out_hbm, idx_v, *rest):
    bufs = list(rest[:NBUF])
    out_v = rest[NBUF]
    sems = list(rest[NBUF + 1:NBUF + 1 + NBUF])
    _pool_kernel(x_hbm, table_hbm, out_hbm, idx_v, bufs, out_v, sems)


def _mlp_kernel(h_ref, w1_ref, b1_ref, w2_ref, b2_ref, w3_ref, b3_ref, o_ref):
    dn = (((1,), (1,)), ((), ()))
    h = h_ref[...]
    z = lax.dot_general(h, w1_ref[...], dn, preferred_element_type=jnp.float32)
    z = jnp.maximum(z + b1_ref[...], 0.0)
    z = lax.dot_general(z, w2_ref[...], dn, preferred_element_type=jnp.float32)
    z = jnp.maximum(z + b2_ref[...], 0.0)
    z = lax.dot_general(z, w3_ref[...], dn, preferred_element_type=jnp.float32)
    o_ref[...] = z + b3_ref[...]


def kernel(x, table, W1, b1, W2, b2, W3, b3):
    h = _pool(x, table.astype(jnp.bfloat16))
    w1p = W1[:, _PERM] * jnp.float32(1.0 / L)
    return pl.pallas_call(
        _mlp_kernel,
        out_shape=jax.ShapeDtypeStruct((B, EMB), jnp.float32),
    )(h, w1p, b1.reshape(1, -1), W2, b2.reshape(1, -1), W3, b3.reshape(1, -1))
